# Initial kernel scaffold; baseline (speedup 1.0000x reference)
import jax
import jax.numpy as jnp
from jax.experimental import pallas as pl


def kernel(h, src_ids, rel_ids, Wq, Wk, Wv, relvec):
    raise NotImplementedError("write your pallas kernel here")


# speedup vs baseline: 4.2340x; 4.2340x over previous
import functools
import math

import jax
import jax.numpy as jnp
from jax import lax
from jax.experimental import pallas as pl
from jax.experimental.pallas import tpu as pltpu
from jax.experimental.pallas import tpu_sc as plsc

N = 10000
DEG = 32
H = 128
NH = 4
NR = 16
DH = H // NH
E = N * DEG

NC = 2
NS = 16
NW = NC * NS
E_PER_W = E // NW
CHUNK = 80
N_CHUNKS = E_PER_W // CHUNK


def _sc_gather(h, src_flat):
    mesh = plsc.VectorSubcoreMesh(core_axis_name="c", subcore_axis_name="s")

    @functools.partial(
        pl.kernel,
        mesh=mesh,
        out_type=jax.ShapeDtypeStruct((E, H), jnp.float32),
        scratch_types=[
            pltpu.VMEM((E_PER_W,), jnp.int32),
            pltpu.VMEM((CHUNK, H), jnp.float32),
            pltpu.SemaphoreType.DMA,
        ],
    )
    def gather_kernel(h_hbm, idx_hbm, out_hbm, idx_v, rows_v, sem):
        wid = lax.axis_index("s") * NC + lax.axis_index("c")
        wbase = wid * E_PER_W
        pltpu.sync_copy(idx_hbm.at[pl.ds(wbase, E_PER_W)], idx_v)

        def body(c, carry):
            base = c * CHUNK
            pltpu.async_copy(
                h_hbm.at[idx_v.at[pl.ds(base, CHUNK)]], rows_v, sem
            ).wait()
            pltpu.sync_copy(rows_v, out_hbm.at[pl.ds(wbase + base, CHUNK)])
            return carry

        lax.fori_loop(0, N_CHUNKS, body, 0)

    return gather_kernel(h, src_flat)


def _tc_body(h_ref, msg_ref, rel_ref, wq_ref, wk_ref, wv_ref, rv_ref, out_ref,
             *, blk):
    eb = blk * DEG
    hb = h_ref[...]
    msg = msg_ref[...]
    rel = rel_ref[...]

    oh = (rel == lax.broadcasted_iota(jnp.int32, (eb, NR), 1)).astype(jnp.float32)
    msg = msg + lax.dot_general(
        oh, rv_ref[...], (((1,), (0,)), ((), ())),
        preferred_element_type=jnp.float32)
    msg = jnp.where(msg >= 0, msg, 0.25 * msg)

    q = lax.dot_general(hb, wq_ref[...], (((1,), (1,)), ((), ())),
                        preferred_element_type=jnp.float32)
    kk = lax.dot_general(msg, wk_ref[...], (((1,), (1,)), ((), ())),
                         preferred_element_type=jnp.float32)
    vv = lax.dot_general(msg, wv_ref[...], (((1,), (1,)), ((), ())),
                         preferred_element_type=jnp.float32)

    S = (lax.broadcasted_iota(jnp.int32, (H, NH), 0) // DH
         == lax.broadcasted_iota(jnp.int32, (H, NH), 1)).astype(jnp.float32)

    p = (kk.reshape(blk, DEG, H) * q[:, None, :]).reshape(eb, H)
    scores = lax.dot_general(p, S, (((1,), (0,)), ((), ())),
                             preferred_element_type=jnp.float32)
    s3 = scores.reshape(blk, DEG, NH) * (1.0 / math.sqrt(DH))
    m = jnp.max(s3, axis=1, keepdims=True)
    e = jnp.exp(s3 - m)
    a = e / jnp.sum(e, axis=1, keepdims=True)

    ab = lax.dot_general(a.reshape(eb, NH), S, (((1,), (1,)), ((), ())),
                         preferred_element_type=jnp.float32)
    red = jnp.sum((ab * vv).reshape(blk, DEG, H), axis=1)

    x = jnp.where(red > 0, red, jnp.exp(red) - 1.0)
    out_ref[...] = hb + x


def _tc_attention(h, msg_raw, rel_flat, Wq, Wk, Wv, relvec, blk=400):
    grid = N // blk
    return pl.pallas_call(
        functools.partial(_tc_body, blk=blk),
        grid=(grid,),
        in_specs=[
            pl.BlockSpec((blk, H), lambda i: (i, 0)),
            pl.BlockSpec((blk * DEG, H), lambda i: (i, 0)),
            pl.BlockSpec((blk * DEG, 1), lambda i: (i, 0)),
            pl.BlockSpec((H, H), lambda i: (0, 0)),
            pl.BlockSpec((H, H), lambda i: (0, 0)),
            pl.BlockSpec((H, H), lambda i: (0, 0)),
            pl.BlockSpec((NR, H), lambda i: (0, 0)),
        ],
        out_specs=pl.BlockSpec((blk, H), lambda i: (i, 0)),
        out_shape=jax.ShapeDtypeStruct((N, H), jnp.float32),
    )(h, msg_raw, rel_flat, Wq, Wk, Wv, relvec)


def kernel(h, src_ids, rel_ids, Wq, Wk, Wv, relvec):
    src_flat = src_ids.astype(jnp.int32).reshape(E)
    rel_flat = rel_ids.astype(jnp.int32).reshape(E, 1)
    msg_raw = _sc_gather(h, src_flat)
    return _tc_attention(h, msg_raw, rel_flat, Wq, Wk, Wv, relvec)


# speedup vs baseline: 4.6142x; 1.0898x over previous
import functools
import math

import jax
import jax.numpy as jnp
from jax import lax
from jax.experimental import pallas as pl
from jax.experimental.pallas import tpu as pltpu
from jax.experimental.pallas import tpu_sc as plsc

N = 10000
DEG = 32
H = 128
NH = 4
NR = 16
DH = H // NH
E = N * DEG

NC = 2
NS = 16
NW = NC * NS
E_PER_W = E // NW
CHUNK = 80
N_CHUNKS = E_PER_W // CHUNK
NBUF = 5
ROUNDS = N_CHUNKS // NBUF


def _sc_gather(h, src_flat):
    mesh = plsc.VectorSubcoreMesh(core_axis_name="c", subcore_axis_name="s")

    @functools.partial(
        pl.kernel,
        mesh=mesh,
        out_type=jax.ShapeDtypeStruct((E, H), jnp.float32),
        scratch_types=[
            pltpu.VMEM((E_PER_W,), jnp.int32),
        ]
        + [pltpu.VMEM((CHUNK, H), jnp.float32) for _ in range(NBUF)]
        + [pltpu.SemaphoreType.DMA for _ in range(2 * NBUF)],
    )
    def gather_kernel(h_hbm, idx_hbm, out_hbm, idx_v, *bufs_sems):
        rows = bufs_sems[:NBUF]
        gsem = bufs_sems[NBUF:2 * NBUF]
        ssem = bufs_sems[2 * NBUF:]
        wid = lax.axis_index("s") * NC + lax.axis_index("c")
        wbase = wid * E_PER_W
        pltpu.sync_copy(idx_hbm.at[pl.ds(wbase, E_PER_W)], idx_v)

        def g_start(c, b):
            pltpu.make_async_copy(
                h_hbm.at[idx_v.at[pl.ds(c * CHUNK, CHUNK)]], rows[b], gsem[b]
            ).start()

        def g_wait(b):
            pltpu.make_async_copy(
                h_hbm.at[idx_v.at[pl.ds(0, CHUNK)]], rows[b], gsem[b]
            ).wait()

        def s_start(c, b):
            pltpu.make_async_copy(
                rows[b], out_hbm.at[pl.ds(wbase + c * CHUNK, CHUNK)], ssem[b]
            ).start()

        def s_wait(b):
            pltpu.make_async_copy(
                rows[b], out_hbm.at[pl.ds(wbase, CHUNK)], ssem[b]
            ).wait()

        for b in range(NBUF):
            g_start(b, b)

        def body(r, carry):
            for b in range(NBUF):
                c = r * NBUF + b
                g_wait(b)
                s_start(c, b)

                @pl.when(r < ROUNDS - 1)
                def _():
                    s_wait(b)
                    g_start(c + NBUF, b)
            return carry

        lax.fori_loop(0, ROUNDS, body, 0)
        for b in range(NBUF):
            s_wait(b)

    return gather_kernel(h, src_flat)


def _tc_body(h_ref, msg_ref, rel_ref, wq_ref, wk_ref, wv_ref, rv_ref, out_ref,
             *, blk):
    eb = blk * DEG
    hb = h_ref[...]
    msg = msg_ref[...]
    rel = rel_ref[...]

    oh = (rel == lax.broadcasted_iota(jnp.int32, (eb, NR), 1)).astype(jnp.float32)
    msg = msg + lax.dot_general(
        oh, rv_ref[...], (((1,), (0,)), ((), ())),
        preferred_element_type=jnp.float32)
    msg = jnp.where(msg >= 0, msg, 0.25 * msg)

    q = lax.dot_general(hb, wq_ref[...], (((1,), (1,)), ((), ())),
                        preferred_element_type=jnp.float32)
    kk = lax.dot_general(msg, wk_ref[...], (((1,), (1,)), ((), ())),
                         preferred_element_type=jnp.float32)
    vv = lax.dot_general(msg, wv_ref[...], (((1,), (1,)), ((), ())),
                         preferred_element_type=jnp.float32)

    S = (lax.broadcasted_iota(jnp.int32, (H, NH), 0) // DH
         == lax.broadcasted_iota(jnp.int32, (H, NH), 1)).astype(jnp.float32)

    p = (kk.reshape(blk, DEG, H) * q[:, None, :]).reshape(eb, H)
    scores = lax.dot_general(p, S, (((1,), (0,)), ((), ())),
                             preferred_element_type=jnp.float32)
    s3 = scores.reshape(blk, DEG, NH) * (1.0 / math.sqrt(DH))
    m = jnp.max(s3, axis=1, keepdims=True)
    e = jnp.exp(s3 - m)
    a = e / jnp.sum(e, axis=1, keepdims=True)

    ab = lax.dot_general(a.reshape(eb, NH), S, (((1,), (1,)), ((), ())),
                         preferred_element_type=jnp.float32)
    red = jnp.sum((ab * vv).reshape(blk, DEG, H), axis=1)

    x = jnp.where(red > 0, red, jnp.exp(red) - 1.0)
    out_ref[...] = hb + x


def _tc_attention(h, msg_raw, rel_flat, Wq, Wk, Wv, relvec, blk=400):
    grid = N // blk
    return pl.pallas_call(
        functools.partial(_tc_body, blk=blk),
        grid=(grid,),
        in_specs=[
            pl.BlockSpec((blk, H), lambda i: (i, 0)),
            pl.BlockSpec((blk * DEG, H), lambda i: (i, 0)),
            pl.BlockSpec((blk * DEG, 1), lambda i: (i, 0)),
            pl.BlockSpec((H, H), lambda i: (0, 0)),
            pl.BlockSpec((H, H), lambda i: (0, 0)),
            pl.BlockSpec((H, H), lambda i: (0, 0)),
            pl.BlockSpec((NR, H), lambda i: (0, 0)),
        ],
        out_specs=pl.BlockSpec((blk, H), lambda i: (i, 0)),
        out_shape=jax.ShapeDtypeStruct((N, H), jnp.float32),
    )(h, msg_raw, rel_flat, Wq, Wk, Wv, relvec)


def kernel(h, src_ids, rel_ids, Wq, Wk, Wv, relvec):
    src_flat = src_ids.astype(jnp.int32).reshape(E)
    rel_flat = rel_ids.astype(jnp.int32).reshape(E, 1)
    msg_raw = _sc_gather(h, src_flat)
    return _tc_attention(h, msg_raw, rel_flat, Wq, Wk, Wv, relvec)
